# Initial kernel scaffold; baseline (speedup 1.0000x reference)
import jax
import jax.numpy as jnp
from jax.experimental import pallas as pl


def kernel(x, embeddings):
    raise NotImplementedError("write your pallas kernel here")


# speedup vs baseline: 1.4049x; 1.4049x over previous
import functools

import jax
import jax.numpy as jnp
from jax import lax
from jax.experimental import pallas as pl
from jax.experimental.pallas import tpu as pltpu
from jax.experimental.pallas import tpu_sc as plsc

_TM = 512


def _dist_argmin_body(x2_ref, e2_ref, x_ref, emb_ref, idx_ref):
    x = x_ref[...]
    emb = emb_ref[...]
    xe = lax.dot_general(
        x, emb, (((1,), (1,)), ((), ())),
        preferred_element_type=jnp.float32)
    d = (x2_ref[...] + e2_ref[...]) - 2.0 * xe
    m = jnp.min(d, axis=1, keepdims=True)
    iot = lax.broadcasted_iota(jnp.int32, d.shape, 1)
    cand = jnp.where(d == m, iot, jnp.int32(d.shape[1]))
    idx_ref[...] = jnp.min(cand, axis=1, keepdims=True)


def _nearest_code_indices(x2, e2, xf, embeddings):
    m, d = xf.shape
    k = embeddings.shape[0]
    grid = (m // _TM,)
    return pl.pallas_call(
        _dist_argmin_body,
        grid=grid,
        in_specs=[
            pl.BlockSpec((_TM, 1), lambda i: (i, 0)),
            pl.BlockSpec((1, k), lambda i: (0, 0)),
            pl.BlockSpec((_TM, d), lambda i: (i, 0)),
            pl.BlockSpec((k, d), lambda i: (0, 0)),
        ],
        out_specs=pl.BlockSpec((_TM, 1), lambda i: (i, 0)),
        out_shape=jax.ShapeDtypeStruct((m, 1), jnp.int32),
    )(x2, e2, xf, embeddings)


def _gather_rows(table, idx):
    b = idx.shape[0]
    d = table.shape[1]
    info = plsc.get_sparse_core_info()
    nw = info.num_cores * info.num_subcores
    b_per_w = b // nw
    mesh = plsc.VectorSubcoreMesh(core_axis_name="c", subcore_axis_name="s")

    @functools.partial(
        pl.kernel, mesh=mesh,
        out_type=jax.ShapeDtypeStruct((b, d), jnp.float32),
        scratch_types=[
            pltpu.VMEM((b_per_w,), jnp.int32),
            pltpu.VMEM((b_per_w, d), jnp.float32),
            pltpu.SemaphoreType.DMA,
        ],
    )
    def gather_kernel(table_hbm, idx_hbm, out_hbm, idx_v, rows_v, sem):
        wid = lax.axis_index("s") * info.num_cores + lax.axis_index("c")
        base = wid * b_per_w
        pltpu.sync_copy(idx_hbm.at[pl.ds(base, b_per_w)], idx_v)
        pltpu.async_copy(table_hbm.at[idx_v], rows_v, sem).wait()
        pltpu.sync_copy(rows_v, out_hbm.at[pl.ds(base, b_per_w)])

    return gather_kernel(table, idx)


def kernel(x, embeddings):
    bsz, t, d = x.shape
    m = bsz * t
    xf = x.reshape(m, d)
    x2 = jnp.sum(xf * xf, axis=-1, keepdims=True)
    e2 = jnp.sum(embeddings * embeddings, axis=-1)[None]
    ind = _nearest_code_indices(x2, e2, xf, embeddings)
    emb = _gather_rows(embeddings, ind.reshape(m))
    emb = emb.reshape(bsz, t, d)
    return (emb, emb)


# speedup vs baseline: 1.4071x; 1.0016x over previous
import functools

import jax
import jax.numpy as jnp
from jax import lax
from jax.experimental import pallas as pl
from jax.experimental.pallas import tpu as pltpu
from jax.experimental.pallas import tpu_sc as plsc

_TM = 512


def _dist_argmin_body(x2_ref, e2_ref, x_ref, emb_ref, idx_ref):
    x = x_ref[...]
    emb = emb_ref[...]
    xe = lax.dot_general(
        x, emb, (((1,), (1,)), ((), ())),
        preferred_element_type=jnp.float32)
    d = (x2_ref[...] + e2_ref[...]) - 2.0 * xe
    m = jnp.min(d, axis=1, keepdims=True)
    iot = lax.broadcasted_iota(jnp.int32, d.shape, 1)
    cand = jnp.where(d == m, iot, jnp.int32(d.shape[1]))
    idx_ref[...] = jnp.min(cand, axis=1, keepdims=True)


def _nearest_code_indices(x2, e2, xf, embeddings):
    m, d = xf.shape
    k = embeddings.shape[0]
    grid = (m // _TM,)
    return pl.pallas_call(
        _dist_argmin_body,
        grid=grid,
        in_specs=[
            pl.BlockSpec((_TM, 1), lambda i: (i, 0)),
            pl.BlockSpec((1, k), lambda i: (0, 0)),
            pl.BlockSpec((_TM, d), lambda i: (i, 0)),
            pl.BlockSpec((k, d), lambda i: (0, 0)),
        ],
        out_specs=pl.BlockSpec((_TM, 1), lambda i: (i, 0)),
        out_shape=jax.ShapeDtypeStruct((m, 1), jnp.int32),
        compiler_params=pltpu.CompilerParams(
            dimension_semantics=("parallel",)),
    )(x2, e2, xf, embeddings)


def _gather_rows(table, idx):
    b = idx.shape[0]
    d = table.shape[1]
    info = plsc.get_sparse_core_info()
    nw = info.num_cores * info.num_subcores
    b_per_w = b // nw
    mesh = plsc.VectorSubcoreMesh(core_axis_name="c", subcore_axis_name="s")

    @functools.partial(
        pl.kernel, mesh=mesh,
        out_type=jax.ShapeDtypeStruct((b, d), jnp.float32),
        scratch_types=[
            pltpu.VMEM((b_per_w,), jnp.int32),
            pltpu.VMEM((b_per_w, d), jnp.float32),
            pltpu.SemaphoreType.DMA,
        ],
    )
    def gather_kernel(table_hbm, idx_hbm, out_hbm, idx_v, rows_v, sem):
        wid = lax.axis_index("s") * info.num_cores + lax.axis_index("c")
        base = wid * b_per_w
        pltpu.sync_copy(idx_hbm.at[pl.ds(base, b_per_w)], idx_v)
        pltpu.async_copy(table_hbm.at[idx_v], rows_v, sem).wait()
        pltpu.sync_copy(rows_v, out_hbm.at[pl.ds(base, b_per_w)])

    return gather_kernel(table, idx)


def kernel(x, embeddings):
    bsz, t, d = x.shape
    m = bsz * t
    xf = x.reshape(m, d)
    x2 = jnp.sum(xf * xf, axis=-1, keepdims=True)
    e2 = jnp.sum(embeddings * embeddings, axis=-1)[None]
    ind = _nearest_code_indices(x2, e2, xf, embeddings)
    emb = _gather_rows(embeddings, ind.reshape(m))
    emb = emb.reshape(bsz, t, d)
    return (emb, emb)
